# baseline (speedup 1.0000x reference)
import functools
import math

import jax
import jax.numpy as jnp
from jax.experimental import pallas as pl
from jax.experimental.pallas import tpu as pltpu

_TOP_K = 28
_BIG = 1e9
_NEG = -1e30


def _bdot(a, b, dims):
    return jax.lax.dot_general(a.astype(jnp.bfloat16), b.astype(jnp.bfloat16),
                               (dims, ((), ())),
                               preferred_element_type=jnp.float32)


def _proj_kernel(re_ref, qh_ref, wa_ref, ba_ref, wq_ref, bq_ref,
                 wk_ref, bk_ref, wv_ref, bv_ref,
                 q_ref, k_ref, v_ref):
    ta = _bdot(re_ref[...], wa_ref[...], ((1,), (0,))) + ba_ref[...]
    k_ref[...] = _bdot(ta, wk_ref[...], ((1,), (0,))) + bk_ref[...]
    v_ref[...] = _bdot(ta, wv_ref[...], ((1,), (0,))) + bv_ref[...]
    q_ref[...] = _bdot(qh_ref[...], wq_ref[...], ((1,), (0,))) + bq_ref[...]


def _attn_kernel(q_ref, mb_ref, k_ref, gij_ref, v_ref, out_ref, *, inv_scale):
    f32 = jnp.float32
    s = _bdot(q_ref[...], k_ref[...], ((1,), (1,)))
    ps = jax.lax.dot_general(mb_ref[...], gij_ref[...],
                             (((1,), (0,)), ((), ())),
                             preferred_element_type=f32)
    z = s * inv_scale + (ps - 2.0) * _BIG

    rows, kp = z.shape
    nch = kp // 128
    gsz = 5
    ngr = (nch + gsz - 1) // gsz
    groups = []
    for j in range(ngr):
        gm = z[:, j * gsz * 128:(j * gsz + 1) * 128]
        for c in range(j * gsz + 1, min((j + 1) * gsz, nch)):
            gm = jnp.maximum(gm, z[:, c * 128:(c + 1) * 128])
        groups.append(gm)
    g = jnp.concatenate(groups, axis=1)

    m1 = jnp.max(g, axis=1, keepdims=True)
    active = (m1 > -_BIG * 0.5).astype(f32)

    def body(_, vv):
        return jnp.max(jnp.where(g < vv, g, _NEG), axis=1, keepdims=True)

    tau = jax.lax.fori_loop(1, _TOP_K, body, m1)

    cnt = jnp.sum((z >= tau).astype(f32), axis=1, keepdims=True)

    def fix_cond(carry):
        _, _, prog = carry
        return prog

    def fix_body(carry):
        u, c, _ = carry
        nxt = jnp.min(jnp.where(z > u, z, -_NEG), axis=1, keepdims=True)
        c2 = jnp.sum((z >= nxt).astype(f32), axis=1, keepdims=True)
        move = jnp.logical_and(c > float(_TOP_K), c2 >= float(_TOP_K))
        u2 = jnp.where(move, nxt, u)
        c3 = jnp.where(move, c2, c)
        return u2, c3, jnp.any(move)

    tau, cnt, _ = jax.lax.while_loop(
        fix_cond, fix_body, (tau, cnt, jnp.any(cnt > float(_TOP_K))))

    e = jnp.where(z >= tau, jnp.exp(z - m1), 0.0)
    w = e * (active / jnp.sum(e, axis=1, keepdims=True))
    out_ref[...] = _bdot(w, v_ref[...], ((1,), (0,)))


def kernel(query_hidden, surviving_mask, rel_embs, f_i, f_j,
           Wa, ba, Wq, bq, Wk, bk, Wv, bv):
    f32 = jnp.float32
    B, T, H = query_hidden.shape
    F = surviving_mask.shape[-1]
    KREL, D = rel_embs.shape
    rows = B * T
    inv_scale = 1.0 / math.sqrt(H)

    KP = ((KREL + 127) // 128) * 128
    FP = ((F + 127) // 128) * 128
    TILE = 512 if rows % 512 == 0 else rows
    grid = rows // TILE

    qh2 = query_hidden.reshape(rows, H)
    re_p = jnp.pad(rel_embs, ((0, KP - KREL), (0, 0)))

    fidx = jnp.arange(FP, dtype=jnp.int32)[:, None]
    gij = ((f_i[None, :].astype(jnp.int32) == fidx).astype(f32)
           + (f_j[None, :].astype(jnp.int32) == fidx).astype(f32))
    gij = jnp.pad(gij, ((0, 0), (0, KP - KREL))).astype(jnp.bfloat16)
    mb = jnp.pad(surviving_mask.reshape(rows, F).astype(f32),
                 ((0, 0), (0, FP - F))).astype(jnp.bfloat16)

    q, k, v = pl.pallas_call(
        _proj_kernel,
        out_shape=(
            jax.ShapeDtypeStruct((rows, H), f32),
            jax.ShapeDtypeStruct((KP, H), f32),
            jax.ShapeDtypeStruct((KP, H), f32),
        ),
    )(re_p, qh2, Wa, ba.reshape(1, H), Wq, bq.reshape(1, H),
      Wk, bk.reshape(1, H), Wv, bv.reshape(1, H))

    out = pl.pallas_call(
        functools.partial(_attn_kernel, inv_scale=inv_scale),
        grid=(grid,),
        in_specs=[
            pl.BlockSpec((TILE, H), lambda i: (i, 0)),
            pl.BlockSpec((TILE, FP), lambda i: (i, 0)),
            pl.BlockSpec((KP, H), lambda i: (0, 0)),
            pl.BlockSpec((FP, KP), lambda i: (0, 0)),
            pl.BlockSpec((KP, H), lambda i: (0, 0)),
        ],
        out_specs=pl.BlockSpec((TILE, H), lambda i: (i, 0)),
        out_shape=jax.ShapeDtypeStruct((rows, H), f32),
        compiler_params=pltpu.CompilerParams(
            dimension_semantics=("parallel",)),
    )(q, mb, k, gij, v)

    return out.reshape(B, T, H)


# speedup vs baseline: 1.0078x; 1.0078x over previous
import functools
import math

import jax
import jax.numpy as jnp
from jax.experimental import pallas as pl
from jax.experimental.pallas import tpu as pltpu

_TOP_K = 28
_BIG = 1e9
_NEG = -1e30


def _bdot(a, b, dims):
    return jax.lax.dot_general(a.astype(jnp.bfloat16), b.astype(jnp.bfloat16),
                               (dims, ((), ())),
                               preferred_element_type=jnp.float32)


def _proj_kernel(re_ref, qh_ref, wa_ref, ba_ref, wq_ref, bq_ref,
                 wk_ref, bk_ref, wv_ref, bv_ref,
                 q_ref, k_ref, v_ref):
    ta = _bdot(re_ref[...], wa_ref[...], ((1,), (0,))) + ba_ref[...]
    k_ref[...] = _bdot(ta, wk_ref[...], ((1,), (0,))) + bk_ref[...]
    v_ref[...] = _bdot(ta, wv_ref[...], ((1,), (0,))) + bv_ref[...]
    q_ref[...] = _bdot(qh_ref[...], wq_ref[...], ((1,), (0,))) + bq_ref[...]


def _attn_kernel(q_ref, mb_ref, k_ref, gij_ref, v_ref, out_ref, w_ref,
                 *, inv_scale, nsteps):
    f32 = jnp.float32
    i = pl.program_id(0)

    out_ref[...] = jax.lax.dot_general(w_ref[...], v_ref[...].astype(jnp.bfloat16),
                                       (((1,), (0,)), ((), ())),
                                       preferred_element_type=f32)

    @pl.when(i < nsteps - 1)
    def _compute():
        _attn_tile(q_ref, mb_ref, k_ref, gij_ref, w_ref, inv_scale=inv_scale)


def _attn_tile(q_ref, mb_ref, k_ref, gij_ref, w_ref, *, inv_scale):
    f32 = jnp.float32
    s = _bdot(q_ref[...], k_ref[...], ((1,), (1,)))
    ps = jax.lax.dot_general(mb_ref[...], gij_ref[...],
                             (((1,), (0,)), ((), ())),
                             preferred_element_type=f32)
    z = s * inv_scale + (ps - 2.0) * _BIG

    rows, kp = z.shape
    nch = kp // 128
    gsz = 5
    ngr = (nch + gsz - 1) // gsz
    groups = []
    for j in range(ngr):
        gm = z[:, j * gsz * 128:(j * gsz + 1) * 128]
        for c in range(j * gsz + 1, min((j + 1) * gsz, nch)):
            gm = jnp.maximum(gm, z[:, c * 128:(c + 1) * 128])
        groups.append(gm)
    g = jnp.concatenate(groups, axis=1)

    m1 = jnp.max(g, axis=1, keepdims=True)
    active = (m1 > -_BIG * 0.5).astype(f32)

    def body(_, vv):
        return jnp.max(jnp.where(g < vv, g, _NEG), axis=1, keepdims=True)

    tau = jax.lax.fori_loop(1, _TOP_K, body, m1)

    cnt = jnp.sum((z >= tau).astype(f32), axis=1, keepdims=True)

    def fix_cond(carry):
        _, _, prog = carry
        return prog

    def fix_body(carry):
        u, c, _ = carry
        nxt = jnp.min(jnp.where(z > u, z, -_NEG), axis=1, keepdims=True)
        c2 = jnp.sum((z >= nxt).astype(f32), axis=1, keepdims=True)
        move = jnp.logical_and(c > float(_TOP_K), c2 >= float(_TOP_K))
        u2 = jnp.where(move, nxt, u)
        c3 = jnp.where(move, c2, c)
        return u2, c3, jnp.any(move)

    tau, cnt, _ = jax.lax.while_loop(
        fix_cond, fix_body, (tau, cnt, jnp.any(cnt > float(_TOP_K))))

    e = jnp.where(z >= tau, jnp.exp(z - m1), 0.0)
    w = e * (active / jnp.sum(e, axis=1, keepdims=True))
    w_ref[...] = w.astype(jnp.bfloat16)


def kernel(query_hidden, surviving_mask, rel_embs, f_i, f_j,
           Wa, ba, Wq, bq, Wk, bk, Wv, bv):
    f32 = jnp.float32
    B, T, H = query_hidden.shape
    F = surviving_mask.shape[-1]
    KREL, D = rel_embs.shape
    rows = B * T
    inv_scale = 1.0 / math.sqrt(H)

    KP = ((KREL + 127) // 128) * 128
    FP = ((F + 127) // 128) * 128
    TILE = 512 if rows % 512 == 0 else rows
    grid = rows // TILE

    qh2 = query_hidden.reshape(rows, H)
    re_p = jnp.pad(rel_embs, ((0, KP - KREL), (0, 0)))

    fidx = jnp.arange(FP, dtype=jnp.int32)[:, None]
    gij = ((f_i[None, :].astype(jnp.int32) == fidx).astype(f32)
           + (f_j[None, :].astype(jnp.int32) == fidx).astype(f32))
    gij = jnp.pad(gij, ((0, 0), (0, KP - KREL))).astype(jnp.bfloat16)
    mb = jnp.pad(surviving_mask.reshape(rows, F).astype(f32),
                 ((0, 0), (0, FP - F))).astype(jnp.bfloat16)

    q, k, v = pl.pallas_call(
        _proj_kernel,
        out_shape=(
            jax.ShapeDtypeStruct((rows, H), f32),
            jax.ShapeDtypeStruct((KP, H), f32),
            jax.ShapeDtypeStruct((KP, H), f32),
        ),
    )(re_p, qh2, Wa, ba.reshape(1, H), Wq, bq.reshape(1, H),
      Wk, bk.reshape(1, H), Wv, bv.reshape(1, H))

    nsteps = grid + 1
    last = grid - 1
    out = pl.pallas_call(
        functools.partial(_attn_kernel, inv_scale=inv_scale, nsteps=nsteps),
        grid=(nsteps,),
        in_specs=[
            pl.BlockSpec((TILE, H), lambda i: (jnp.minimum(i, last), 0)),
            pl.BlockSpec((TILE, FP), lambda i: (jnp.minimum(i, last), 0)),
            pl.BlockSpec((KP, H), lambda i: (0, 0)),
            pl.BlockSpec((FP, KP), lambda i: (0, 0)),
            pl.BlockSpec((KP, H), lambda i: (0, 0)),
        ],
        out_specs=pl.BlockSpec((TILE, H),
                               lambda i: (jnp.maximum(i - 1, 0), 0)),
        out_shape=jax.ShapeDtypeStruct((rows, H), f32),
        scratch_shapes=[pltpu.VMEM((TILE, KP), jnp.bfloat16)],
    )(q, mb, k, gij, v)

    return out.reshape(B, T, H)


# speedup vs baseline: 1.0765x; 1.0682x over previous
import functools
import math

import jax
import jax.numpy as jnp
from jax.experimental import pallas as pl
from jax.experimental.pallas import tpu as pltpu

_TOP_K = 28
_BIG = 1e9
_NEG = -1e30


def _bdot(a, b, dims):
    return jax.lax.dot_general(a.astype(jnp.bfloat16), b.astype(jnp.bfloat16),
                               (dims, ((), ())),
                               preferred_element_type=jnp.float32)


def _proj_kernel(re_ref, qh_ref, wa_ref, ba_ref, wq_ref, bq_ref,
                 wk_ref, bk_ref, wv_ref, bv_ref,
                 q_ref, k_ref, v_ref):
    ta = _bdot(re_ref[...], wa_ref[...], ((1,), (0,))) + ba_ref[...]
    k_ref[...] = _bdot(ta, wk_ref[...], ((1,), (0,))) + bk_ref[...]
    v_ref[...] = _bdot(ta, wv_ref[...], ((1,), (0,))) + bv_ref[...]
    q_ref[...] = _bdot(qh_ref[...], wq_ref[...], ((1,), (0,))) + bq_ref[...]


def _attn_kernel(q_ref, mb_ref, k_ref, gij_ref, v_ref, out_ref, w_ref, sc_ref,
                 *, inv_scale, nsteps):
    f32 = jnp.float32
    i = pl.program_id(0)

    out_ref[...] = jax.lax.dot_general(w_ref[...], v_ref[...].astype(jnp.bfloat16),
                                       (((1,), (0,)), ((), ())),
                                       preferred_element_type=f32) * sc_ref[...]

    @pl.when(i < nsteps - 1)
    def _compute():
        _attn_tile(q_ref, mb_ref, k_ref, gij_ref, w_ref, sc_ref,
                   inv_scale=inv_scale)


def _attn_tile(q_ref, mb_ref, k_ref, gij_ref, w_ref, sc_ref, *, inv_scale):
    f32 = jnp.float32
    s = _bdot(q_ref[...], k_ref[...], ((1,), (1,)))
    ps = jax.lax.dot_general(mb_ref[...], gij_ref[...],
                             (((1,), (0,)), ((), ())),
                             preferred_element_type=f32)
    z = s * inv_scale + (ps - 2.0) * _BIG

    rows, kp = z.shape
    nch = kp // 128
    gsz = 5
    ngr = (nch + gsz - 1) // gsz
    groups = []
    for j in range(ngr):
        gm = z[:, j * gsz * 128:(j * gsz + 1) * 128]
        for c in range(j * gsz + 1, min((j + 1) * gsz, nch)):
            gm = jnp.maximum(gm, z[:, c * 128:(c + 1) * 128])
        groups.append(gm)
    g = jnp.concatenate(groups, axis=1)

    m1 = jnp.max(g, axis=1, keepdims=True)
    active = (m1 > -_BIG * 0.5).astype(f32)

    def body(_, vv):
        return jnp.max(jnp.where(g < vv, g, _NEG), axis=1, keepdims=True)

    tau = jax.lax.fori_loop(1, _TOP_K, body, m1)

    cnt = jnp.full_like(tau, _BIG)

    def fix_cond(carry):
        _, _, prog = carry
        return prog

    def fix_body(carry):
        u, c, _ = carry
        nxt = jnp.min(jnp.where(z > u, z, -_NEG), axis=1, keepdims=True)
        c2 = jnp.sum((z >= nxt).astype(f32), axis=1, keepdims=True)
        move = jnp.logical_and(c > float(_TOP_K), c2 >= float(_TOP_K))
        u2 = jnp.where(move, nxt, u)
        c3 = jnp.where(move, c2, c)
        return u2, c3, jnp.any(move)

    tau, cnt, _ = jax.lax.while_loop(
        fix_cond, fix_body, (tau, cnt, jnp.array(True)))

    e = jnp.where(z >= tau, jnp.exp(z - m1), 0.0)
    sc_ref[...] = active / jnp.sum(e, axis=1, keepdims=True)
    w_ref[...] = e.astype(jnp.bfloat16)


def kernel(query_hidden, surviving_mask, rel_embs, f_i, f_j,
           Wa, ba, Wq, bq, Wk, bk, Wv, bv):
    f32 = jnp.float32
    B, T, H = query_hidden.shape
    F = surviving_mask.shape[-1]
    KREL, D = rel_embs.shape
    rows = B * T
    inv_scale = 1.0 / math.sqrt(H)

    KP = ((KREL + 127) // 128) * 128
    FP = ((F + 127) // 128) * 128
    TILE = 512 if rows % 512 == 0 else rows
    grid = rows // TILE

    qh2 = query_hidden.reshape(rows, H)
    re_p = jnp.pad(rel_embs, ((0, KP - KREL), (0, 0)))

    fidx = jnp.arange(FP, dtype=jnp.int32)[:, None]
    gij = ((f_i[None, :].astype(jnp.int32) == fidx).astype(f32)
           + (f_j[None, :].astype(jnp.int32) == fidx).astype(f32))
    gij = jnp.pad(gij, ((0, 0), (0, KP - KREL))).astype(jnp.bfloat16)
    mb = jnp.pad(surviving_mask.reshape(rows, F).astype(f32),
                 ((0, 0), (0, FP - F))).astype(jnp.bfloat16)

    q, k, v = pl.pallas_call(
        _proj_kernel,
        out_shape=(
            jax.ShapeDtypeStruct((rows, H), f32),
            jax.ShapeDtypeStruct((KP, H), f32),
            jax.ShapeDtypeStruct((KP, H), f32),
        ),
    )(re_p, qh2, Wa, ba.reshape(1, H), Wq, bq.reshape(1, H),
      Wk, bk.reshape(1, H), Wv, bv.reshape(1, H))

    nsteps = grid + 1
    last = grid - 1
    out = pl.pallas_call(
        functools.partial(_attn_kernel, inv_scale=inv_scale, nsteps=nsteps),
        grid=(nsteps,),
        in_specs=[
            pl.BlockSpec((TILE, H), lambda i: (jnp.minimum(i, last), 0)),
            pl.BlockSpec((TILE, FP), lambda i: (jnp.minimum(i, last), 0)),
            pl.BlockSpec((KP, H), lambda i: (0, 0)),
            pl.BlockSpec((FP, KP), lambda i: (0, 0)),
            pl.BlockSpec((KP, H), lambda i: (0, 0)),
        ],
        out_specs=pl.BlockSpec((TILE, H),
                               lambda i: (jnp.maximum(i - 1, 0), 0)),
        out_shape=jax.ShapeDtypeStruct((rows, H), f32),
        scratch_shapes=[pltpu.VMEM((TILE, KP), jnp.bfloat16),
                        pltpu.VMEM((TILE, 1), f32)],
    )(q, mb, k, gij, v)

    return out.reshape(B, T, H)
